# baseline (speedup 1.0000x reference)
import functools

import jax
import jax.numpy as jnp
from jax import lax
from jax.experimental import pallas as pl
from jax.experimental.pallas import tpu as pltpu
from jax.experimental.pallas import tpu_sc as plsc

N_FIELDS = 26
VOCAB = 100000
EMB = 20
B = 16384

NC, NS = 2, 16
NW = NC * NS
BW = B // NW
WIN = 4 * EMB
VWIN = VOCAB // 4
CHUNK = 128
NSB = BW // CHUNK
XPAD = 32
NUNIT = N_FIELDS * NSB
GROUPS = CHUNK * EMB // 16
OUT_D = N_FIELDS * EMB

VB = 2000
NBLK = N_FIELDS * (VOCAB // VB)
UMAX = (NBLK + NW - 1) // NW
BWORDS = VB * EMB
PAIRS = VB // 4

_MESH = plsc.VectorSubcoreMesh(core_axis_name="c", subcore_axis_name="s")
_CP = pltpu.CompilerParams(use_tc_tiling_on_sc=False, needs_layout_passes=False)


@functools.partial(
    pl.kernel,
    out_type=jax.ShapeDtypeStruct((N_FIELDS, VWIN, WIN), jnp.float32),
    mesh=_MESH,
    scratch_types=[
        pltpu.VMEM((BWORDS,), jnp.float32),
        pltpu.VMEM((BWORDS,), jnp.float32),
        pltpu.VMEM((PAIRS, WIN), jnp.float32),
        pltpu.VMEM((WIN,), jnp.int32),
        pltpu.SemaphoreType.DMA,
        pltpu.SemaphoreType.DMA,
    ],
    compiler_params=_CP,
)
def _transpose_kernel(t_lin, t_win, tb0, tb1, wbuf, src80, sem0, sem1):
    wid = lax.axis_index("s") * NC + lax.axis_index("c")
    iota = lax.iota(jnp.int32, 16)

    for g in range(WIN // 16):
        c = iota + 16 * g
        half = (c >= 40).astype(jnp.int32)
        c40 = c - 40 * half
        q = (c40 >= EMB).astype(jnp.int32)
        cm20 = c40 - EMB * q
        src80[pl.ds(16 * g, 16)] = cm20 * VB + 2 * half + q

    def block_id(u):
        bid = u * NW + wid
        return jnp.minimum(bid, NBLK - 1)

    def fire(u, tb, sem):
        bid = block_id(u)
        f = lax.div(bid, VOCAB // VB)
        vs = lax.rem(bid, VOCAB // VB) * VB
        return [
            pltpu.async_copy(
                t_lin.at[f, d, pl.ds(vs, VB)],
                tb.at[pl.ds(d * VB, VB)], sem)
            for d in range(EMB)
        ]

    def transpose_out(u, tb, descs):
        bid = block_id(u)
        f = lax.div(bid, VOCAB // VB)
        c0 = lax.rem(bid, VOCAB // VB)
        for d in descs:
            d.wait()

        def rowbody(r, carry):
            off = jnp.full((16,), 4 * r, jnp.int32)
            for g in range(WIN // 16):
                srcv = src80[pl.ds(16 * g, 16)] + off
                data = plsc.load_gather(tb, [srcv])
                wbuf[r, pl.ds(16 * g, 16)] = data
            return carry

        lax.fori_loop(0, PAIRS, rowbody, 0, unroll=4)
        pltpu.sync_copy(wbuf, t_win.at[f, pl.ds(c0 * PAIRS, PAIRS)])

    def body(k, carry):
        d0 = fire(2 * k, tb0, sem0)
        d1 = fire(2 * k + 1, tb1, sem1)
        transpose_out(2 * k, tb0, d0)
        transpose_out(2 * k + 1, tb1, d1)
        return carry

    lax.fori_loop(0, (UMAX + 1) // 2, body, 0, unroll=False)


@functools.partial(
    pl.kernel,
    out_type=jax.ShapeDtypeStruct((B, OUT_D), jnp.float32),
    mesh=_MESH,
    scratch_types=[
        pltpu.VMEM((CHUNK, XPAD), jnp.int32),
        pltpu.VMEM((N_FIELDS, BW), jnp.int32),
        pltpu.VMEM((N_FIELDS, BW), jnp.int32),
        pltpu.VMEM((CHUNK, WIN), jnp.float32),
        pltpu.VMEM((CHUNK, WIN), jnp.float32),
        pltpu.VMEM((CHUNK, OUT_D), jnp.float32),
        pltpu.VMEM((CHUNK * EMB + 16,), jnp.int32),
        pltpu.VMEM((CHUNK * EMB + 16,), jnp.int32),
        pltpu.SemaphoreType.DMA,
        pltpu.SemaphoreType.DMA,
    ],
    compiler_params=_CP,
)
def _gather_kernel(t3, x_hbm, out_hbm, xb, idx_v, corr_v, rows0, rows1,
                   big, row_t, pw_t, sem0, sem1):
    wid = lax.axis_index("s") * NC + lax.axis_index("c")
    b0 = wid * BW
    iota = lax.iota(jnp.int32, 16)

    def const_body(j, carry):
        row_t[pl.ds(j * EMB, 16)] = jnp.full((16,), j, jnp.int32)
        row_t[pl.ds(j * EMB + 16, 16)] = jnp.full((16,), j, jnp.int32)
        pw_t[pl.ds(j * EMB, 16)] = iota
        pw_t[pl.ds(j * EMB + 16, 16)] = iota + 16
        return carry

    lax.fori_loop(0, CHUNK, const_body, 0, unroll=4)

    def prep_sb(sb, carry):
        pltpu.sync_copy(x_hbm.at[pl.ds(b0 + sb * CHUNK, CHUNK)], xb)

        def prep_f(f, carry2):
            fvec = jnp.full((16,), f, jnp.int32)

            def prep_g(g, carry3):
                v = plsc.load_gather(xb, [iota + g * 16, fvec])
                idx_v[f, pl.ds(sb * CHUNK + g * 16, 16)] = v >> 2
                corr_v[f, pl.ds(sb * CHUNK + g * 16, 16)] = (v & 3) * EMB
                return carry3

            return lax.fori_loop(0, CHUNK // 16, prep_g, carry2, unroll=4)

        return lax.fori_loop(0, N_FIELDS, prep_f, carry, unroll=False)

    lax.fori_loop(0, NSB, prep_sb, 0, unroll=False)

    def fire(u, rows, sem):
        f = lax.rem(u, N_FIELDS)
        sb = lax.div(u, N_FIELDS)
        return pltpu.async_copy(
            t3.at[f].at[idx_v.at[f, pl.ds(sb * CHUNK, CHUNK)]], rows, sem)

    def repack_and_flush(u, rows):
        f = lax.rem(u, N_FIELDS)
        sb = lax.div(u, N_FIELDS)
        fvec = jnp.full((16,), f, jnp.int32)
        sboff = jnp.full((16,), sb * CHUNK, jnp.int32)

        def group(g, carry2):
            rowv = row_t[pl.ds(g * 16, 16)]
            pwv = pw_t[pl.ds(g * 16, 16)]
            corr = plsc.load_gather(corr_v, [fvec, sboff + rowv])
            data = plsc.load_gather(rows, [rowv, corr + pwv])
            plsc.store_scatter(big, [rowv, pwv + f * EMB], data)
            return carry2

        lax.fori_loop(0, GROUPS, group, 0, unroll=8)

        @pl.when(f == N_FIELDS - 1)
        def _():
            pltpu.sync_copy(big, out_hbm.at[pl.ds(b0 + sb * CHUNK, CHUNK)])

    def body(k, carry):
        u0 = 2 * k
        d0 = fire(u0, rows0, sem0)
        d1 = fire(u0 + 1, rows1, sem1)
        d0.wait()
        repack_and_flush(u0, rows0)
        d1.wait()
        repack_and_flush(u0 + 1, rows1)
        return carry

    lax.fori_loop(0, NUNIT // 2, body, 0, unroll=False)


def kernel(x, table):
    t_lin = jnp.transpose(table, (0, 2, 1))
    t_win = _transpose_kernel(t_lin)
    xpad = jnp.pad(x, ((0, 0), (0, XPAD - N_FIELDS)))
    return _gather_kernel(t_win, xpad)


# speedup vs baseline: 2.0457x; 2.0457x over previous
import functools

import jax
import jax.numpy as jnp
from jax import lax
from jax.experimental import pallas as pl
from jax.experimental.pallas import tpu as pltpu
from jax.experimental.pallas import tpu_sc as plsc

N_FIELDS = 26
VOCAB = 100000
EMB = 20
B = 16384

NC, NS = 2, 16
NW = NC * NS
BW = B // NW
WIN = 2 * EMB
VWIN = VOCAB // 2
CHUNK = 128
NSB = BW // CHUNK
XPAD = 32
NUNIT = N_FIELDS * NSB
GROUPS = CHUNK * EMB // 16
OUT_D = N_FIELDS * EMB


@functools.partial(
    pl.kernel,
    out_type=jax.ShapeDtypeStruct((B, OUT_D), jnp.float32),
    mesh=plsc.VectorSubcoreMesh(core_axis_name="c", subcore_axis_name="s"),
    scratch_types=[
        pltpu.VMEM((BW, XPAD), jnp.int32),
        pltpu.VMEM((N_FIELDS, BW), jnp.int32),
        pltpu.VMEM((N_FIELDS, BW), jnp.int32),
        pltpu.VMEM((CHUNK, WIN), jnp.float32),
        pltpu.VMEM((CHUNK, WIN), jnp.float32),
        pltpu.VMEM((CHUNK, OUT_D), jnp.float32),
        pltpu.VMEM((CHUNK * EMB + 16,), jnp.int32),
        pltpu.VMEM((CHUNK * EMB + 16,), jnp.int32),
        pltpu.SemaphoreType.DMA,
        pltpu.SemaphoreType.DMA,
    ],
    compiler_params=pltpu.CompilerParams(
        use_tc_tiling_on_sc=False, needs_layout_passes=False),
)
def _emb_kernel(t3, x_hbm, out_hbm, xb, idx_v, corr_v, rows0, rows1,
                big, row_t, pw_t, sem0, sem1):
    wid = lax.axis_index("s") * NC + lax.axis_index("c")
    b0 = wid * BW
    pltpu.sync_copy(x_hbm.at[pl.ds(b0, BW)], xb)

    iota = lax.iota(jnp.int32, 16)

    def const_body(j, carry):
        row_t[pl.ds(j * EMB, 16)] = jnp.full((16,), j, jnp.int32)
        row_t[pl.ds(j * EMB + 16, 16)] = jnp.full((16,), j, jnp.int32)
        pw_t[pl.ds(j * EMB, 16)] = iota
        pw_t[pl.ds(j * EMB + 16, 16)] = iota + 16
        return carry

    lax.fori_loop(0, CHUNK, const_body, 0, unroll=False)

    def prep_f(f, carry):
        def prep_g(g, carry2):
            rows = iota + g * 16
            v = plsc.load_gather(xb, [rows, jnp.full((16,), f, jnp.int32)])
            idx_v[f, pl.ds(g * 16, 16)] = v >> 1
            corr_v[f, pl.ds(g * 16, 16)] = (v & 1) * EMB
            return carry2
        return lax.fori_loop(0, BW // 16, prep_g, carry, unroll=False)

    lax.fori_loop(0, N_FIELDS, prep_f, 0, unroll=False)

    def fire(u, rows, sem):
        f = lax.rem(u, N_FIELDS)
        sb = lax.div(u, N_FIELDS)
        return pltpu.async_copy(
            t3.at[f].at[idx_v.at[f, pl.ds(sb * CHUNK, CHUNK)]], rows, sem)

    def repack_and_flush(u, rows):
        f = lax.rem(u, N_FIELDS)
        sb = lax.div(u, N_FIELDS)
        fvec = jnp.full((16,), f, jnp.int32)
        sboff = jnp.full((16,), sb * CHUNK, jnp.int32)

        def group(g, carry2):
            rowv = row_t[pl.ds(g * 16, 16)]
            pwv = pw_t[pl.ds(g * 16, 16)]
            corr = plsc.load_gather(corr_v, [fvec, sboff + rowv])
            data = plsc.load_gather(rows, [rowv, corr + pwv])
            plsc.store_scatter(big, [rowv, pwv + f * EMB], data)
            return carry2

        lax.fori_loop(0, GROUPS, group, 0, unroll=False)

        @pl.when(f == N_FIELDS - 1)
        def _():
            pltpu.sync_copy(big, out_hbm.at[pl.ds(b0 + sb * CHUNK, CHUNK)])

    def body(k, carry):
        u0 = 2 * k
        d0 = fire(u0, rows0, sem0)
        d1 = fire(u0 + 1, rows1, sem1)
        d0.wait()
        repack_and_flush(u0, rows0)
        d1.wait()
        repack_and_flush(u0 + 1, rows1)
        return carry

    lax.fori_loop(0, NUNIT // 2, body, 0, unroll=False)


def kernel(x, table):
    t3 = table.reshape(N_FIELDS, VWIN, WIN)
    xpad = jnp.pad(x, ((0, 0), (0, XPAD - N_FIELDS)))
    return _emb_kernel(t3, xpad)
